# baseline (speedup 1.0000x reference)
import functools

import jax
import jax.numpy as jnp
from jax.experimental import pallas as pl
from jax.experimental.pallas import tpu as pltpu


def _round_up(x, m):
    return ((x + m - 1) // m) * m


def _pad(a, target_shape):
    widths = [(0, t - s) for s, t in zip(a.shape, target_shape)]
    if all(w == (0, 0) for w in widths):
        return a
    return jnp.pad(a, widths)


def _fused_kernel(x_ref, w1_ref, b1_ref, w2_ref, b2_ref, wp_ref, bp_ref,
                  o_ref, acc_ref, *, inv_t):
    t = pl.program_id(1)

    @pl.when(t == 0)
    def _():
        acc_ref[...] = jnp.zeros_like(acc_ref)

    acc_ref[...] += jnp.sum(x_ref[...].astype(jnp.float32), axis=1)

    @pl.when(t == pl.num_programs(1) - 1)
    def _():
        pooled = (acc_ref[...] * inv_t).astype(w1_ref.dtype)
        h = jnp.dot(pooled, w1_ref[...], preferred_element_type=jnp.float32)
        h = jnp.maximum(h + b1_ref[...], 0.0).astype(w2_ref.dtype)
        h = jnp.dot(h, w2_ref[...], preferred_element_type=jnp.float32)
        h = (h + b2_ref[...]).astype(wp_ref.dtype)
        out = jnp.dot(h, wp_ref[...], preferred_element_type=jnp.float32)
        o_ref[...] = (out + bp_ref[...]).astype(o_ref.dtype)


def _resident(shape):
    return pl.BlockSpec(shape, lambda b, t: (0, 0), pipeline_mode=pl.Buffered(1))


def kernel(x, w1, b1, w2, b2, w_proj, b_proj):
    B, T, D_in = x.shape
    D_out = w_proj.shape[1]
    D_in_p = _round_up(D_in, 128)
    D_out_p = _round_up(D_out, 128)

    if B >= 16:
        TB = _round_up((B + 1) // 2, 8)
    else:
        TB = _round_up(max(B, 1), 8)
    B_pad = _round_up(B, TB)

    itemsize = jnp.dtype(x.dtype).itemsize
    TT = max(8, (12 * 1024 * 1024) // (TB * D_in_p * itemsize) // 8 * 8)
    TT = min(TT, _round_up(T, 8))
    T_pad = _round_up(T, TT)

    wdt = jnp.bfloat16
    x_p = _pad(x, (B_pad, T_pad, D_in_p))
    w1p = _pad(w1, (D_in_p, D_in_p)).astype(wdt)
    b1p = _pad(b1.reshape(1, -1), (1, D_in_p)).astype(jnp.float32)
    w2p = _pad(w2, (D_in_p, D_in_p)).astype(wdt)
    b2p = _pad(b2.reshape(1, -1), (1, D_in_p)).astype(jnp.float32)
    wpp = _pad(w_proj, (D_in_p, D_out_p)).astype(wdt)
    bpp = _pad(b_proj.reshape(1, -1), (1, D_out_p)).astype(jnp.float32)

    grid = (B_pad // TB, T_pad // TT)
    weight_bytes = (2 * D_in_p * D_in_p + D_in_p * D_out_p) * 2
    bytes_accessed = (x_p.size * itemsize + weight_bytes
                      + B_pad * D_out_p * itemsize)
    flops = (B_pad * T_pad * D_in_p + 4 * B_pad * D_in_p * D_in_p
             + 2 * B_pad * D_in_p * D_out_p)

    out = pl.pallas_call(
        functools.partial(_fused_kernel, inv_t=1.0 / T),
        out_shape=jax.ShapeDtypeStruct((B_pad, D_out_p), x.dtype),
        grid=grid,
        in_specs=[
            pl.BlockSpec((TB, TT, D_in_p), lambda b, t: (b, t, 0)),
            _resident((D_in_p, D_in_p)),
            _resident((1, D_in_p)),
            _resident((D_in_p, D_in_p)),
            _resident((1, D_in_p)),
            _resident((D_in_p, D_out_p)),
            _resident((1, D_out_p)),
        ],
        out_specs=pl.BlockSpec((TB, D_out_p), lambda b, t: (b, 0)),
        scratch_shapes=[pltpu.VMEM((TB, D_in_p), jnp.float32)],
        compiler_params=pltpu.CompilerParams(
            dimension_semantics=("parallel", "arbitrary"),
            vmem_limit_bytes=56 * 1024 * 1024),
        cost_estimate=pl.CostEstimate(
            flops=int(flops), transcendentals=0,
            bytes_accessed=int(bytes_accessed)),
    )(x_p, w1p, b1p, w2p, b2p, wpp, bpp)

    return out[:B, None, :D_out]


# speedup vs baseline: 1.1719x; 1.1719x over previous
import jax
import jax.numpy as jnp
from jax.experimental import pallas as pl
from jax.experimental.pallas import tpu as pltpu


def _round_up(x, m):
    return ((x + m - 1) // m) * m


def _pad(a, target_shape):
    widths = [(0, t - s) for s, t in zip(a.shape, target_shape)]
    if all(w == (0, 0) for w in widths):
        return a
    return jnp.pad(a, widths)


def _pool_kernel(x_ref, o_ref, acc_ref):
    t = pl.program_id(1)

    @pl.when(t == 0)
    def _():
        acc_ref[...] = jnp.zeros_like(acc_ref)

    acc_ref[...] += jnp.sum(x_ref[...].astype(jnp.float32), axis=1)

    @pl.when(t == pl.num_programs(1) - 1)
    def _():
        o_ref[...] = acc_ref[...]


def _head_kernel(p_ref, w1_ref, b1_ref, w2_ref, b2_ref, wp_ref, bp_ref,
                 o_ref, *, inv_t):
    pooled = (p_ref[...] * inv_t).astype(jnp.bfloat16)
    h = jnp.dot(pooled, w1_ref[...].astype(jnp.bfloat16),
                preferred_element_type=jnp.float32)
    h = jnp.maximum(h + b1_ref[...], 0.0).astype(jnp.bfloat16)
    h = jnp.dot(h, w2_ref[...].astype(jnp.bfloat16),
                preferred_element_type=jnp.float32)
    h = (h + b2_ref[...]).astype(jnp.bfloat16)
    out = jnp.dot(h, wp_ref[...].astype(jnp.bfloat16),
                  preferred_element_type=jnp.float32)
    o_ref[...] = (out + bp_ref[...]).astype(o_ref.dtype)


def kernel(x, w1, b1, w2, b2, w_proj, b_proj):
    import functools

    B, T, D_in = x.shape
    D_out = w_proj.shape[1]
    D_in_p = _round_up(D_in, 128)
    D_out_p = _round_up(D_out, 128)
    itemsize = jnp.dtype(x.dtype).itemsize

    if B >= 16:
        TB = _round_up((B + 1) // 2, 8)
    else:
        TB = _round_up(max(B, 1), 8)
    B_pad = _round_up(B, TB)

    TT = max(8, (12 * 1024 * 1024) // (TB * D_in_p * itemsize) // 8 * 8)
    TT = min(TT, _round_up(T, 8))
    T_pad = _round_up(T, TT)

    x_p = _pad(x, (B_pad, T_pad, D_in_p))
    w1p = _pad(w1, (D_in_p, D_in_p))
    b1p = _pad(b1.reshape(1, -1), (1, D_in_p))
    w2p = _pad(w2, (D_in_p, D_in_p))
    b2p = _pad(b2.reshape(1, -1), (1, D_in_p))
    wpp = _pad(w_proj, (D_in_p, D_out_p))
    bpp = _pad(b_proj.reshape(1, -1), (1, D_out_p))

    grid = (B_pad // TB, T_pad // TT)
    pooled = pl.pallas_call(
        _pool_kernel,
        out_shape=jax.ShapeDtypeStruct((B_pad, D_in_p), jnp.float32),
        grid=grid,
        in_specs=[pl.BlockSpec((TB, TT, D_in_p), lambda b, t: (b, t, 0))],
        out_specs=pl.BlockSpec((TB, D_in_p), lambda b, t: (b, 0)),
        scratch_shapes=[pltpu.VMEM((TB, D_in_p), jnp.float32)],
        compiler_params=pltpu.CompilerParams(
            dimension_semantics=("parallel", "arbitrary"),
            vmem_limit_bytes=56 * 1024 * 1024),
        cost_estimate=pl.CostEstimate(
            flops=int(B_pad * T_pad * D_in_p), transcendentals=0,
            bytes_accessed=int(x_p.size * itemsize + B_pad * D_in_p * 4)),
    )(x_p)

    head_flops = (4 * B_pad * D_in_p * D_in_p + 2 * B_pad * D_in_p * D_out_p)
    head_bytes = ((2 * D_in_p * D_in_p + D_in_p * D_out_p) * 4
                  + B_pad * D_in_p * 4 + B_pad * D_out_p * itemsize)
    out = pl.pallas_call(
        functools.partial(_head_kernel, inv_t=1.0 / T),
        out_shape=jax.ShapeDtypeStruct((B_pad, D_out_p), x.dtype),
        grid=(1,),
        in_specs=[
            pl.BlockSpec((B_pad, D_in_p), lambda i: (0, 0)),
            pl.BlockSpec((D_in_p, D_in_p), lambda i: (0, 0)),
            pl.BlockSpec((1, D_in_p), lambda i: (0, 0)),
            pl.BlockSpec((D_in_p, D_in_p), lambda i: (0, 0)),
            pl.BlockSpec((1, D_in_p), lambda i: (0, 0)),
            pl.BlockSpec((D_in_p, D_out_p), lambda i: (0, 0)),
            pl.BlockSpec((1, D_out_p), lambda i: (0, 0)),
        ],
        out_specs=pl.BlockSpec((B_pad, D_out_p), lambda i: (0, 0)),
        compiler_params=pltpu.CompilerParams(
            dimension_semantics=("arbitrary",),
            vmem_limit_bytes=56 * 1024 * 1024),
        cost_estimate=pl.CostEstimate(
            flops=int(head_flops), transcendentals=0,
            bytes_accessed=int(head_bytes)),
    )(pooled, w1p, b1p, w2p, b2p, wpp, bpp)

    return out[:B, None, :D_out]


# speedup vs baseline: 1.2362x; 1.0548x over previous
import functools

import jax
import jax.numpy as jnp
from jax.experimental import pallas as pl
from jax.experimental.pallas import tpu as pltpu


def _round_up(x, m):
    return ((x + m - 1) // m) * m


def _pad(a, target_shape):
    widths = [(0, t - s) for s, t in zip(a.shape, target_shape)]
    if all(w == (0, 0) for w in widths):
        return a
    return jnp.pad(a, widths)


def _pool_kernel(x_ref, o_ref, acc_ref):
    t = pl.program_id(1)

    @pl.when(t == 0)
    def _():
        acc_ref[...] = jnp.zeros_like(acc_ref)

    acc_ref[...] += jnp.sum(x_ref[...].astype(jnp.float32), axis=1)

    @pl.when(t == pl.num_programs(1) - 1)
    def _():
        o_ref[...] = acc_ref[...]


def _head_kernel(p_ref, w1_ref, b1_ref, w2_ref, b2_ref, wp_ref, bp_ref,
                 o_ref, *, inv_t):
    pooled = (p_ref[...] * inv_t).astype(jnp.bfloat16)
    h = jnp.dot(pooled, w1_ref[...].astype(jnp.bfloat16),
                preferred_element_type=jnp.float32)
    h = jnp.maximum(h + b1_ref[...], 0.0).astype(jnp.bfloat16)
    h = jnp.dot(h, w2_ref[...].astype(jnp.bfloat16),
                preferred_element_type=jnp.float32)
    h = (h + b2_ref[...]).astype(jnp.bfloat16)
    out = jnp.dot(h, wp_ref[...].astype(jnp.bfloat16),
                  preferred_element_type=jnp.float32)
    o_ref[...] = (out + bp_ref[...]).astype(o_ref.dtype)


def _resident(shape, index_map):
    return pl.BlockSpec(shape, index_map, pipeline_mode=pl.Buffered(1))


def kernel(x, w1, b1, w2, b2, w_proj, b_proj):
    B, T, D_in = x.shape
    D_out = w_proj.shape[1]
    D_in_p = _round_up(D_in, 128)
    D_out_p = _round_up(D_out, 128)
    itemsize = jnp.dtype(x.dtype).itemsize

    if B >= 16:
        TB = _round_up((B + 1) // 2, 8)
    else:
        TB = _round_up(max(B, 1), 8)
    B_pad = _round_up(B, TB)

    TT = max(8, (6 * 1024 * 1024) // (TB * D_in_p * itemsize) // 8 * 8)
    TT = min(TT, _round_up(T, 8))
    T_pad = _round_up(T, TT)

    x_p = _pad(x, (B_pad, T_pad, D_in_p))
    w1p = _pad(w1, (D_in_p, D_in_p))
    b1p = _pad(b1.reshape(1, -1), (1, D_in_p))
    w2p = _pad(w2, (D_in_p, D_in_p))
    b2p = _pad(b2.reshape(1, -1), (1, D_in_p))
    wpp = _pad(w_proj, (D_in_p, D_out_p))
    bpp = _pad(b_proj.reshape(1, -1), (1, D_out_p))

    grid = (B_pad // TB, T_pad // TT)
    pooled = pl.pallas_call(
        _pool_kernel,
        out_shape=jax.ShapeDtypeStruct((B_pad, D_in_p), jnp.float32),
        grid=grid,
        in_specs=[pl.BlockSpec((TB, TT, D_in_p), lambda b, t: (b, t, 0))],
        out_specs=pl.BlockSpec((TB, D_in_p), lambda b, t: (b, 0)),
        scratch_shapes=[pltpu.VMEM((TB, D_in_p), jnp.float32)],
        compiler_params=pltpu.CompilerParams(
            dimension_semantics=("parallel", "arbitrary"),
            vmem_limit_bytes=56 * 1024 * 1024),
        cost_estimate=pl.CostEstimate(
            flops=int(B_pad * T_pad * D_in_p), transcendentals=0,
            bytes_accessed=int(x_p.size * itemsize + B_pad * D_in_p * 4)),
    )(x_p)

    n_j = 4 if D_out_p % (4 * 128) == 0 else (2 if D_out_p % 256 == 0 else 1)
    TJ = D_out_p // n_j
    head_flops = (4 * B_pad * D_in_p * D_in_p + 2 * B_pad * D_in_p * D_out_p)
    head_bytes = ((2 * D_in_p * D_in_p + D_in_p * D_out_p) * 4
                  + B_pad * D_in_p * 4 + B_pad * D_out_p * itemsize)
    out = pl.pallas_call(
        functools.partial(_head_kernel, inv_t=1.0 / T),
        out_shape=jax.ShapeDtypeStruct((B_pad, D_out_p), x.dtype),
        grid=(n_j,),
        in_specs=[
            _resident((B_pad, D_in_p), lambda j: (0, 0)),
            _resident((D_in_p, D_in_p), lambda j: (0, 0)),
            _resident((1, D_in_p), lambda j: (0, 0)),
            _resident((D_in_p, D_in_p), lambda j: (0, 0)),
            _resident((1, D_in_p), lambda j: (0, 0)),
            pl.BlockSpec((D_in_p, TJ), lambda j: (0, j)),
            pl.BlockSpec((1, TJ), lambda j: (0, j)),
        ],
        out_specs=pl.BlockSpec((B_pad, TJ), lambda j: (0, j)),
        compiler_params=pltpu.CompilerParams(
            dimension_semantics=("parallel",),
            vmem_limit_bytes=56 * 1024 * 1024),
        cost_estimate=pl.CostEstimate(
            flops=int(head_flops), transcendentals=0,
            bytes_accessed=int(head_bytes)),
    )(pooled, w1p, b1p, w2p, b2p, wpp, bpp)

    return out[:B, None, :D_out]
